# baseline (speedup 1.0000x reference)
import functools

import jax
import jax.numpy as jnp
from jax import lax
from jax.experimental import pallas as pl
from jax.experimental.pallas import tpu as pltpu
from jax.experimental.pallas import tpu_sc as plsc

EMB = 128
N_NODES = 10000
N_EDGES = 320000
MAX_DEPTH = 20

_NC, _NS, _L = 2, 16, 16
_NW = _NC * _NS

_EB = 256
_ECH = 39
_EW = _EB * _ECH
_EQ = _EW // 8
_TAIL_BASE = _EW * _NW
_TAIL = N_EDGES - _TAIL_BASE
_OUT3 = N_EDGES // 128

_NP = 12288
_N_PER_W = _NP // _NW

_NB = 2000
_TT_PAD = 104
_DT_PAD = 24


def _mesh():
    return plsc.VectorSubcoreMesh(
        core_axis_name="c", subcore_axis_name="s",
        num_cores=_NC, num_subcores=_NS)


def _comb_body(df_ref, ast_ref, out_ref):
    row = lax.broadcasted_iota(jnp.int32, (32, EMB), 0)
    acc = jnp.zeros((32, EMB), jnp.float32)
    for k in range(8):
        acc = acc + jnp.where(row // 4 == k, df_ref[k:k + 1, :], 0.0)
    for k in range(4):
        acc = acc + jnp.where(row % 4 == k, ast_ref[k:k + 1, :], 0.0)
    out_ref[...] = acc


def _comb_table(edge_df_table, edge_ast_table):
    return pl.pallas_call(
        _comb_body,
        out_shape=jax.ShapeDtypeStruct((32, EMB), jnp.float32),
    )(edge_df_table, edge_ast_table)


def _edge_body(et_hbm, en_hbm, comb_hbm, na_hbm, atab_hbm,
               out_hbm, attr_hbm,
               t_q, n_q, c_v, c_t, ai_v, rows_v, nrows_v, comb_sh,
               isem, gsem0, gsem1, osem, *nsems):
    wid = lax.axis_index("s") * _NC + lax.axis_index("c")
    base_w = wid * _EW
    nbase = wid * _N_PER_W

    pltpu.async_copy(na_hbm.at[pl.ds(nbase, _N_PER_W)], ai_v, isem).wait()
    for h, ns in enumerate(nsems):
        pltpu.async_copy(atab_hbm.at[ai_v.at[pl.ds(h * 96, 96)]],
                         nrows_v.at[pl.ds(h * 96, 96)], ns)

    @pl.when(lax.axis_index("s") == 0)
    def _():
        pltpu.sync_copy(comb_hbm, comb_sh)

    for qr in range(8):
        qoff = qr * _EQ
        a = pltpu.async_copy(et_hbm.at[pl.ds(base_w + qoff, _EQ)], t_q, isem)
        b = pltpu.async_copy(en_hbm.at[pl.ds(base_w + qoff, _EQ)], n_q, isem)
        a.wait()
        b.wait()

        def cbody(i, carry, qoff=qoff):
            sl = pl.ds(i * _L, _L)
            c_v[pl.ds(qoff + i * _L, _L)] = t_q[sl] * 4 + n_q[sl]
            return carry
        lax.fori_loop(0, _EQ // _L, cbody, 0)

    plsc.subcore_barrier()

    @pl.when(wid == _NW - 1)
    def _():
        a = pltpu.async_copy(et_hbm.at[pl.ds(_TAIL_BASE, _TAIL)],
                             t_q.at[pl.ds(0, _TAIL)], isem)
        b = pltpu.async_copy(en_hbm.at[pl.ds(_TAIL_BASE, _TAIL)],
                             n_q.at[pl.ds(0, _TAIL)], isem)
        a.wait()
        b.wait()

        def tbody(i, carry):
            sl = pl.ds(i * _L, _L)
            c_t[sl] = t_q[sl] * 4 + n_q[sl]
            return carry
        lax.fori_loop(0, _TAIL // _L, tbody, 0)
        for h in range(2):
            pltpu.async_copy(comb_sh.at[c_t.at[pl.ds(h * _EB, _EB)]],
                             rows_v.at[0], gsem0).wait()
            pltpu.sync_copy(rows_v.at[0],
                            out_hbm.at[pl.ds(_TAIL_BASE + h * _EB, _EB)])

    def issue_gather(j, slot, gsem):
        pltpu.async_copy(comb_sh.at[c_v.at[pl.ds(_EB * j, _EB)]],
                         rows_v.at[slot], gsem)

    def wait_gather(slot, gsem):
        pltpu.make_async_copy(comb_sh.at[c_v.at[pl.ds(0, _EB)]],
                              rows_v.at[slot], gsem).wait()

    def issue_write(j, slot):
        pltpu.async_copy(rows_v.at[slot],
                         out_hbm.at[pl.ds(base_w + _EB * j, _EB)], osem)

    def wait_write(slot):
        pltpu.make_async_copy(rows_v.at[slot],
                              out_hbm.at[pl.ds(0, _EB)], osem).wait()

    issue_gather(0, 0, gsem0)

    def step(j, sp, sq, gsem_p, gsem_q):
        @pl.when(j >= 1)
        def _():
            wait_write(sq)

        @pl.when(j + 1 < _ECH)
        def _():
            issue_gather(j + 1, sq, gsem_q)

        wait_gather(sp, gsem_p)
        issue_write(j, sp)

    def body(j, carry):
        @pl.when(j % 2 == 0)
        def _():
            step(j, 0, 1, gsem0, gsem1)

        @pl.when(j % 2 == 1)
        def _():
            step(j, 1, 0, gsem1, gsem0)
        return carry

    lax.fori_loop(0, _ECH, body, 0)
    wait_write(1)

    for h, ns in enumerate(nsems):
        pltpu.make_async_copy(atab_hbm.at[ai_v.at[pl.ds(h * 96, 96)]],
                              nrows_v.at[pl.ds(h * 96, 96)], ns).wait()
    pltpu.sync_copy(nrows_v, attr_hbm.at[pl.ds(nbase, _N_PER_W)])


def _edge_sc(et, en, comb, na2, atab):
    f = functools.partial(
        pl.kernel,
        out_type=(jax.ShapeDtypeStruct((N_EDGES, EMB), jnp.float32),
                  jax.ShapeDtypeStruct((_NP, EMB), jnp.float32)),
        mesh=_mesh(),
        scratch_types=[
            pltpu.VMEM((_EQ,), jnp.int32),
            pltpu.VMEM((_EQ,), jnp.int32),
            pltpu.VMEM((_EW,), jnp.int32),
            pltpu.VMEM((_TAIL,), jnp.int32),
            pltpu.VMEM((_N_PER_W,), jnp.int32),
            pltpu.VMEM((2, _EB, EMB), jnp.float32),
            pltpu.VMEM((_N_PER_W, EMB), jnp.float32),
            pltpu.VMEM_SHARED((32, EMB), jnp.float32),
            pltpu.SemaphoreType.DMA,
            pltpu.SemaphoreType.DMA,
            pltpu.SemaphoreType.DMA,
            pltpu.SemaphoreType.DMA,
            pltpu.SemaphoreType.DMA,
            pltpu.SemaphoreType.DMA,
            pltpu.SemaphoreType.DMA,
            pltpu.SemaphoreType.DMA,
        ],
    )(_edge_body)
    return f(et, en, comb, na2, atab)


def _nodesum_body(attr_ref, nt_ref, dp_ref, ttab_ref, dtab_ref, out_ref):
    ntv = nt_ref[0]
    dpv = jnp.minimum(dp_ref[0], MAX_DEPTH)
    kt = lax.broadcasted_iota(jnp.int32, (_TT_PAD, _NB), 0)
    hot_t = (kt == ntv).astype(jnp.float32)
    kd = lax.broadcasted_iota(jnp.int32, (_DT_PAD, _NB), 0)
    hot_d = (kd == dpv).astype(jnp.float32)
    dn = (((0,), (0,)), ((), ()))
    te = lax.dot_general(hot_t, ttab_ref[...], dn,
                         precision=lax.Precision.HIGHEST,
                         preferred_element_type=jnp.float32)
    de = lax.dot_general(hot_d, dtab_ref[...], dn,
                         precision=lax.Precision.HIGHEST,
                         preferred_element_type=jnp.float32)
    out_ref[...] = attr_ref[...] + te + de


def _node_sum_tc(attr_rows, nt3, dp3, ttab_p, dtab_p):
    grid = (N_NODES // _NB,)
    return pl.pallas_call(
        _nodesum_body,
        grid=grid,
        in_specs=[
            pl.BlockSpec((_NB, EMB), lambda i: (i, 0)),
            pl.BlockSpec((1, 1, _NB), lambda i: (i, 0, 0)),
            pl.BlockSpec((1, 1, _NB), lambda i: (i, 0, 0)),
            pl.BlockSpec((_TT_PAD, EMB), lambda i: (0, 0)),
            pl.BlockSpec((_DT_PAD, EMB), lambda i: (0, 0)),
        ],
        out_specs=pl.BlockSpec((_NB, EMB), lambda i: (i, 0)),
        out_shape=jax.ShapeDtypeStruct((N_NODES, EMB), jnp.float32),
    )(attr_rows, nt3, dp3, ttab_p, dtab_p)


def _pad1d(x, total):
    return jnp.pad(x, (0, total - x.shape[0]))


def kernel(nodes, depth, edge_type, edge_name, node_type_table,
           node_attr_table, depth_table, edge_df_table, edge_ast_table):
    na2 = _pad1d(nodes[:, 1].astype(jnp.int32), _NP)
    nt3 = nodes[:, 0].astype(jnp.int32).reshape(N_NODES // _NB, 1, _NB)
    dp3 = depth[:, 0].astype(jnp.int32).reshape(N_NODES // _NB, 1, _NB)
    et = edge_type.astype(jnp.int32)
    en = edge_name.astype(jnp.int32)
    ttab_p = jnp.pad(node_type_table, ((0, _TT_PAD - 100), (0, 0)))
    dtab_p = jnp.pad(depth_table, ((0, _DT_PAD - (MAX_DEPTH + 1)), (0, 0)))

    comb = _comb_table(edge_df_table, edge_ast_table)
    edge_out, attr_rows = _edge_sc(et, en, comb, na2, node_attr_table)
    node_out = _node_sum_tc(attr_rows, nt3, dp3, ttab_p, dtab_p)
    return (node_out, edge_out)


# speedup vs baseline: 1.0126x; 1.0126x over previous
import functools

import jax
import jax.numpy as jnp
from jax import lax
from jax.experimental import pallas as pl
from jax.experimental.pallas import tpu as pltpu
from jax.experimental.pallas import tpu_sc as plsc

EMB = 128
N_NODES = 10000
N_EDGES = 320000
MAX_DEPTH = 20

_NC, _NS, _L = 2, 16, 16
_NW = _NC * _NS

_EB = 256
_ECH = 39
_EW = _EB * _ECH
_EQ = _EW // 8
_TAIL_BASE = _EW * _NW
_TAIL = N_EDGES - _TAIL_BASE
_OUT3 = N_EDGES // 128

_NP = 12288
_N_PER_W = _NP // _NW

_NB = 2000
_TT_PAD = 104
_DT_PAD = 24


def _mesh():
    return plsc.VectorSubcoreMesh(
        core_axis_name="c", subcore_axis_name="s",
        num_cores=_NC, num_subcores=_NS)


def _comb_body(df_ref, ast_ref, out_ref):
    row = lax.broadcasted_iota(jnp.int32, (32, EMB), 0)
    acc = jnp.zeros((32, EMB), jnp.float32)
    for k in range(8):
        acc = acc + jnp.where(row // 4 == k, df_ref[k:k + 1, :], 0.0)
    for k in range(4):
        acc = acc + jnp.where(row % 4 == k, ast_ref[k:k + 1, :], 0.0)
    out_ref[...] = acc


def _comb_table(edge_df_table, edge_ast_table):
    return pl.pallas_call(
        _comb_body,
        out_shape=jax.ShapeDtypeStruct((32, EMB), jnp.float32),
    )(edge_df_table, edge_ast_table)


def _edge_body(et_hbm, en_hbm, comb_hbm, na_hbm, atab_hbm,
               out_hbm, attr_hbm,
               t_q, n_q, c_v, c_t, ai_v, rows_v, nrows_v, comb_sh,
               isem, gsem0, gsem1, osem, nsem):
    wid = lax.axis_index("s") * _NC + lax.axis_index("c")
    base_w = wid * _EW
    nbase = wid * _N_PER_W

    pltpu.async_copy(na_hbm.at[pl.ds(nbase, _N_PER_W)], ai_v, isem).wait()
    pltpu.async_copy(atab_hbm.at[ai_v], nrows_v, nsem)

    @pl.when(lax.axis_index("s") == 0)
    def _():
        pltpu.sync_copy(comb_hbm, comb_sh)

    for qr in range(8):
        qoff = qr * _EQ
        a = pltpu.async_copy(et_hbm.at[pl.ds(base_w + qoff, _EQ)], t_q, isem)
        b = pltpu.async_copy(en_hbm.at[pl.ds(base_w + qoff, _EQ)], n_q, isem)
        a.wait()
        b.wait()

        def cbody(i, carry, qoff=qoff):
            sl = pl.ds(i * _L, _L)
            c_v[pl.ds(qoff + i * _L, _L)] = t_q[sl] * 4 + n_q[sl]
            return carry
        lax.fori_loop(0, _EQ // _L, cbody, 0)

    plsc.subcore_barrier()

    @pl.when(wid == _NW - 1)
    def _():
        a = pltpu.async_copy(et_hbm.at[pl.ds(_TAIL_BASE, _TAIL)],
                             t_q.at[pl.ds(0, _TAIL)], isem)
        b = pltpu.async_copy(en_hbm.at[pl.ds(_TAIL_BASE, _TAIL)],
                             n_q.at[pl.ds(0, _TAIL)], isem)
        a.wait()
        b.wait()

        def tbody(i, carry):
            sl = pl.ds(i * _L, _L)
            c_t[sl] = t_q[sl] * 4 + n_q[sl]
            return carry
        lax.fori_loop(0, _TAIL // _L, tbody, 0)
        for h in range(2):
            pltpu.async_copy(comb_sh.at[c_t.at[pl.ds(h * _EB, _EB)]],
                             rows_v.at[0], gsem0).wait()
            pltpu.sync_copy(rows_v.at[0],
                            out_hbm.at[pl.ds(_TAIL_BASE + h * _EB, _EB)])

    def issue_gather(j, slot, gsem):
        pltpu.async_copy(comb_sh.at[c_v.at[pl.ds(_EB * j, _EB)]],
                         rows_v.at[slot], gsem)

    def wait_gather(slot, gsem):
        pltpu.make_async_copy(comb_sh.at[c_v.at[pl.ds(0, _EB)]],
                              rows_v.at[slot], gsem).wait()

    def issue_write(j, slot):
        pltpu.async_copy(rows_v.at[slot],
                         out_hbm.at[pl.ds(base_w + _EB * j, _EB)], osem)

    def wait_write(slot):
        pltpu.make_async_copy(rows_v.at[slot],
                              out_hbm.at[pl.ds(0, _EB)], osem).wait()

    issue_gather(0, 0, gsem0)

    def step(j, sp, sq, gsem_p, gsem_q):
        @pl.when(j >= 1)
        def _():
            wait_write(sq)

        @pl.when(j + 1 < _ECH)
        def _():
            issue_gather(j + 1, sq, gsem_q)

        wait_gather(sp, gsem_p)
        issue_write(j, sp)

    def body(j, carry):
        @pl.when(j % 2 == 0)
        def _():
            step(j, 0, 1, gsem0, gsem1)

        @pl.when(j % 2 == 1)
        def _():
            step(j, 1, 0, gsem1, gsem0)
        return carry

    lax.fori_loop(0, _ECH, body, 0)
    wait_write(1)

    pltpu.make_async_copy(atab_hbm.at[ai_v], nrows_v, nsem).wait()
    pltpu.sync_copy(nrows_v, attr_hbm.at[pl.ds(nbase, _N_PER_W)])


def _edge_sc(et, en, comb, na2, atab):
    f = functools.partial(
        pl.kernel,
        out_type=(jax.ShapeDtypeStruct((N_EDGES, EMB), jnp.float32),
                  jax.ShapeDtypeStruct((_NP, EMB), jnp.float32)),
        mesh=_mesh(),
        scratch_types=[
            pltpu.VMEM((_EQ,), jnp.int32),
            pltpu.VMEM((_EQ,), jnp.int32),
            pltpu.VMEM((_EW,), jnp.int32),
            pltpu.VMEM((_TAIL,), jnp.int32),
            pltpu.VMEM((_N_PER_W,), jnp.int32),
            pltpu.VMEM((2, _EB, EMB), jnp.float32),
            pltpu.VMEM((_N_PER_W, EMB), jnp.float32),
            pltpu.VMEM_SHARED((32, EMB), jnp.float32),
            pltpu.SemaphoreType.DMA,
            pltpu.SemaphoreType.DMA,
            pltpu.SemaphoreType.DMA,
            pltpu.SemaphoreType.DMA,
            pltpu.SemaphoreType.DMA,
        ],
    )(_edge_body)
    return f(et, en, comb, na2, atab)


def _nodesum_body(attr_ref, nt_ref, dp_ref, ttab_ref, dtab_ref, out_ref):
    ntv = nt_ref[0]
    dpv = jnp.minimum(dp_ref[0], MAX_DEPTH)
    kt = lax.broadcasted_iota(jnp.int32, (_TT_PAD, _NB), 0)
    hot_t = (kt == ntv).astype(jnp.float32)
    kd = lax.broadcasted_iota(jnp.int32, (_DT_PAD, _NB), 0)
    hot_d = (kd == dpv).astype(jnp.float32)
    dn = (((0,), (0,)), ((), ()))
    te = lax.dot_general(hot_t, ttab_ref[...], dn,
                         precision=lax.Precision.HIGHEST,
                         preferred_element_type=jnp.float32)
    de = lax.dot_general(hot_d, dtab_ref[...], dn,
                         precision=lax.Precision.HIGHEST,
                         preferred_element_type=jnp.float32)
    out_ref[...] = attr_ref[...] + te + de


def _node_sum_tc(attr_rows, nt3, dp3, ttab_p, dtab_p):
    grid = (N_NODES // _NB,)
    return pl.pallas_call(
        _nodesum_body,
        grid=grid,
        in_specs=[
            pl.BlockSpec((_NB, EMB), lambda i: (i, 0)),
            pl.BlockSpec((1, 1, _NB), lambda i: (i, 0, 0)),
            pl.BlockSpec((1, 1, _NB), lambda i: (i, 0, 0)),
            pl.BlockSpec((_TT_PAD, EMB), lambda i: (0, 0)),
            pl.BlockSpec((_DT_PAD, EMB), lambda i: (0, 0)),
        ],
        out_specs=pl.BlockSpec((_NB, EMB), lambda i: (i, 0)),
        out_shape=jax.ShapeDtypeStruct((N_NODES, EMB), jnp.float32),
    )(attr_rows, nt3, dp3, ttab_p, dtab_p)


def _pad1d(x, total):
    return jnp.pad(x, (0, total - x.shape[0]))


def kernel(nodes, depth, edge_type, edge_name, node_type_table,
           node_attr_table, depth_table, edge_df_table, edge_ast_table):
    na2 = _pad1d(nodes[:, 1].astype(jnp.int32), _NP)
    nt3 = nodes[:, 0].astype(jnp.int32).reshape(N_NODES // _NB, 1, _NB)
    dp3 = depth[:, 0].astype(jnp.int32).reshape(N_NODES // _NB, 1, _NB)
    et = edge_type.astype(jnp.int32)
    en = edge_name.astype(jnp.int32)
    ttab_p = jnp.pad(node_type_table, ((0, _TT_PAD - 100), (0, 0)))
    dtab_p = jnp.pad(depth_table, ((0, _DT_PAD - (MAX_DEPTH + 1)), (0, 0)))

    comb = _comb_table(edge_df_table, edge_ast_table)
    edge_out, attr_rows = _edge_sc(et, en, comb, na2, node_attr_table)
    node_out = _node_sum_tc(attr_rows, nt3, dp3, ttab_p, dtab_p)
    return (node_out, edge_out)
